# baseline (speedup 1.0000x reference)
import functools

import jax
import jax.numpy as jnp
from jax import lax
from jax.experimental import pallas as pl
from jax.experimental.pallas import tpu as pltpu
from jax.experimental.pallas import tpu_sc as plsc

N = 10000
E = 320000
MAX_NB = 32
ATOM_FDIM = 128
BOND_FDIM = 144
H = 128
STEPS = 3
B = 100

NC = 2
NS = 16
NW = NC * NS
LANES = 16

NPAD = 10240
A_CHUNK = 4
C_CHUNK = 80


def _sc_mesh():
    return plsc.VectorSubcoreMesh(
        core_axis_name="c", subcore_axis_name="s",
        num_cores=NC, num_subcores=NS,
    )


@functools.partial(
    pl.kernel,
    out_type=jax.ShapeDtypeStruct((NPAD, H), jnp.float32),
    mesh=_sc_mesh(),
    scratch_types=[
        pltpu.VMEM((A_CHUNK * MAX_NB,), jnp.int32),
        pltpu.VMEM((A_CHUNK * MAX_NB, H), jnp.float32),
        pltpu.VMEM((A_CHUNK, H), jnp.float32),
        pltpu.SemaphoreType.DMA,
    ],
)
def _sc_gather_sum(msg_hbm, a2b_hbm, out_hbm, idx_v, rows_v, acc_v, sem):
    npw = NPAD // NW
    iters = npw // A_CHUNK
    wid = lax.axis_index("s") * NC + lax.axis_index("c")
    base_atom = wid * npw

    def body(it, carry):
        a0 = base_atom + it * A_CHUNK
        pltpu.sync_copy(a2b_hbm.at[pl.ds(a0 * MAX_NB, A_CHUNK * MAX_NB)], idx_v)
        pltpu.async_copy(msg_hbm.at[idx_v], rows_v, sem).wait()
        for a in range(A_CHUNK):
            for h in range(H // LANES):
                acc = rows_v[a * MAX_NB, pl.ds(h * LANES, LANES)]
                for k in range(1, MAX_NB):
                    acc = acc + rows_v[a * MAX_NB + k, pl.ds(h * LANES, LANES)]
                acc_v[a, pl.ds(h * LANES, LANES)] = acc
        pltpu.sync_copy(acc_v, out_hbm.at[pl.ds(a0, A_CHUNK)])
        return carry

    lax.fori_loop(0, iters, body, 0)


@functools.partial(
    pl.kernel,
    out_type=jax.ShapeDtypeStruct((E, H), jnp.float32),
    mesh=_sc_mesh(),
    scratch_types=[
        pltpu.VMEM((C_CHUNK,), jnp.int32),
        pltpu.VMEM((C_CHUNK,), jnp.int32),
        pltpu.VMEM((C_CHUNK, H), jnp.float32),
        pltpu.VMEM((C_CHUNK, H), jnp.float32),
        pltpu.VMEM((C_CHUNK, H), jnp.float32),
        pltpu.SemaphoreType.DMA,
        pltpu.SemaphoreType.DMA,
    ],
)
def _sc_combine(agg_hbm, msg_hbm, b2a_hbm, b2revb_hbm, out_hbm,
                ia_v, ir_v, qa_v, qr_v, o_v, sem_a, sem_r):
    epw = E // NW
    iters = epw // C_CHUNK
    wid = lax.axis_index("s") * NC + lax.axis_index("c")
    base = wid * epw

    def body(it, carry):
        e0 = base + it * C_CHUNK
        pltpu.sync_copy(b2a_hbm.at[pl.ds(e0, C_CHUNK)], ia_v)
        pltpu.sync_copy(b2revb_hbm.at[pl.ds(e0, C_CHUNK)], ir_v)
        cpa = pltpu.async_copy(agg_hbm.at[ia_v], qa_v, sem_a)
        cpr = pltpu.async_copy(msg_hbm.at[ir_v], qr_v, sem_r)
        cpa.wait()
        cpr.wait()
        for r in range(C_CHUNK):
            for h in range(H // LANES):
                sl = pl.ds(h * LANES, LANES)
                o_v[r, sl] = qa_v[r, sl] - qr_v[r, sl]
        pltpu.sync_copy(o_v, out_hbm.at[pl.ds(e0, C_CHUNK)])
        return carry

    lax.fori_loop(0, iters, body, 0)


def _tc_input_proj(f_bonds, w_i):
    BE = 2000

    def body(x_ref, w_ref, inp_ref, msg_ref):
        acc = jnp.dot(x_ref[...], w_ref[...], preferred_element_type=jnp.float32)
        inp_ref[...] = acc
        msg_ref[...] = jnp.maximum(acc, 0.0)

    return pl.pallas_call(
        body,
        grid=(E // BE,),
        in_specs=[
            pl.BlockSpec((BE, BOND_FDIM), lambda i: (i, 0)),
            pl.BlockSpec((BOND_FDIM, H), lambda i: (0, 0)),
        ],
        out_specs=[
            pl.BlockSpec((BE, H), lambda i: (i, 0)),
            pl.BlockSpec((BE, H), lambda i: (i, 0)),
        ],
        out_shape=[
            jax.ShapeDtypeStruct((E, H), jnp.float32),
            jax.ShapeDtypeStruct((E, H), jnp.float32),
        ],
    )(f_bonds, w_i)


def _tc_update(pre, w_m, inp):
    BE = 2000

    def body(p_ref, w_ref, i_ref, o_ref):
        acc = jnp.dot(p_ref[...], w_ref[...], preferred_element_type=jnp.float32)
        o_ref[...] = jnp.maximum(i_ref[...] + acc, 0.0)

    return pl.pallas_call(
        body,
        grid=(E // BE,),
        in_specs=[
            pl.BlockSpec((BE, H), lambda i: (i, 0)),
            pl.BlockSpec((H, H), lambda i: (0, 0)),
            pl.BlockSpec((BE, H), lambda i: (i, 0)),
        ],
        out_specs=pl.BlockSpec((BE, H), lambda i: (i, 0)),
        out_shape=jax.ShapeDtypeStruct((E, H), jnp.float32),
    )(pre, w_m, inp)


def _tc_readout(f_atoms, a_msg, w_a1, w_a2, b_a, pool):
    def body(f_ref, m_ref, w1_ref, w2_ref, b_ref, p_ref, o_ref):
        h = jnp.dot(f_ref[...], w1_ref[...], preferred_element_type=jnp.float32)
        h = h + jnp.dot(m_ref[...], w2_ref[...], preferred_element_type=jnp.float32)
        h = jnp.maximum(h + b_ref[...], 0.0)
        o_ref[...] = jnp.dot(p_ref[...], h, preferred_element_type=jnp.float32)

    return pl.pallas_call(
        body,
        out_shape=jax.ShapeDtypeStruct((B, H), jnp.float32),
    )(f_atoms, a_msg, w_a1, w_a2, b_a, pool)


def kernel(f_atoms, f_bonds, a2b, b2a, b2revb, W_i, W_m, W_a, b_a):
    a2b_flat = jnp.pad(a2b, ((0, NPAD - N), (0, 0))).reshape(-1)
    w_a1 = W_a[:ATOM_FDIM]
    w_a2 = W_a[ATOM_FDIM:]
    b_a2 = b_a.reshape(1, H)
    apm = N // B
    row = jax.lax.broadcasted_iota(jnp.int32, (B, N), 0)
    col = jax.lax.broadcasted_iota(jnp.int32, (B, N), 1)
    pool = jnp.where(col // apm == row, 1.0 / apm, 0.0).astype(jnp.float32)

    inp, msg = _tc_input_proj(f_bonds, W_i)
    for _ in range(STEPS - 1):
        agg = _sc_gather_sum(msg, a2b_flat)
        pre = _sc_combine(agg, msg, b2a, b2revb)
        msg = _tc_update(pre, W_m, inp)
    a_msg = _sc_gather_sum(msg, a2b_flat)[:N]
    return _tc_readout(f_atoms, a_msg, w_a1, w_a2, b_a2, pool)


# speedup vs baseline: 1.2276x; 1.2276x over previous
import functools

import jax
import jax.numpy as jnp
from jax import lax
from jax.experimental import pallas as pl
from jax.experimental.pallas import tpu as pltpu
from jax.experimental.pallas import tpu_sc as plsc

N = 10000
E = 320000
MAX_NB = 32
ATOM_FDIM = 128
BOND_FDIM = 144
H = 128
STEPS = 3
B = 100

NC = 2
NS = 16
NW = NC * NS
LANES = 16

NPAD = 10368
A_CHUNK = 4
C_CHUNK = 80


def _sc_mesh():
    return plsc.VectorSubcoreMesh(
        core_axis_name="c", subcore_axis_name="s",
        num_cores=NC, num_subcores=NS,
    )


@functools.partial(
    pl.kernel,
    out_type=jax.ShapeDtypeStruct((NPAD, H), jnp.float32),
    mesh=_sc_mesh(),
    scratch_types=[
        pltpu.VMEM((NPAD // NW * MAX_NB,), jnp.int32),
        pltpu.VMEM((A_CHUNK * MAX_NB, H), jnp.float32),
        pltpu.VMEM((A_CHUNK * MAX_NB, H), jnp.float32),
        pltpu.VMEM((A_CHUNK, H), jnp.float32),
        pltpu.SemaphoreType.DMA,
        pltpu.SemaphoreType.DMA,
    ],
)
def _sc_gather_sum(msg_hbm, a2b_hbm, out_hbm, idx_v, rows0_v, rows1_v,
                   acc_v, sem0, sem1):
    npw = NPAD // NW
    iters = npw // A_CHUNK
    nidx = A_CHUNK * MAX_NB
    wid = lax.axis_index("s") * NC + lax.axis_index("c")
    base_atom = wid * npw

    pltpu.sync_copy(a2b_hbm.at[pl.ds(base_atom * MAX_NB, npw * MAX_NB)], idx_v)

    def gather(it, rows_v, sem):
        return pltpu.make_async_copy(
            msg_hbm.at[idx_v.at[pl.ds(it * nidx, nidx)]], rows_v, sem)

    def reduce_store(it, rows_v):
        for a in range(A_CHUNK):
            for h in range(H // LANES):
                sl = pl.ds(h * LANES, LANES)
                acc = rows_v[a * MAX_NB, sl]
                for k in range(1, MAX_NB):
                    acc = acc + rows_v[a * MAX_NB + k, sl]
                acc_v[a, sl] = acc
        pltpu.sync_copy(acc_v,
                        out_hbm.at[pl.ds(base_atom + it * A_CHUNK, A_CHUNK)])

    gather(0, rows0_v, sem0).start()

    def body(p, carry):
        it0 = 2 * p
        gather(it0 + 1, rows1_v, sem1).start()
        gather(it0, rows0_v, sem0).wait()
        reduce_store(it0, rows0_v)
        gather(it0 + 2, rows0_v, sem0).start()
        gather(it0 + 1, rows1_v, sem1).wait()
        reduce_store(it0 + 1, rows1_v)
        return carry

    lax.fori_loop(0, iters // 2, body, 0)
    gather(iters - 1, rows0_v, sem0).wait()
    reduce_store(iters - 1, rows0_v)


@functools.partial(
    pl.kernel,
    out_type=jax.ShapeDtypeStruct((E, H), jnp.float32),
    mesh=_sc_mesh(),
    scratch_types=[
        pltpu.VMEM((E // NW,), jnp.int32),
        pltpu.VMEM((E // NW,), jnp.int32),
        pltpu.VMEM((C_CHUNK, H), jnp.float32),
        pltpu.VMEM((C_CHUNK, H), jnp.float32),
        pltpu.VMEM((C_CHUNK, H), jnp.float32),
        pltpu.VMEM((C_CHUNK, H), jnp.float32),
        pltpu.VMEM((C_CHUNK, H), jnp.float32),
        pltpu.SemaphoreType.DMA,
        pltpu.SemaphoreType.DMA,
        pltpu.SemaphoreType.DMA,
        pltpu.SemaphoreType.DMA,
    ],
)
def _sc_combine(agg_hbm, msg_hbm, b2a_hbm, b2revb_hbm, out_hbm,
                ia_v, ir_v, qa0_v, qr0_v, qa1_v, qr1_v, o_v,
                sem_a0, sem_r0, sem_a1, sem_r1):
    epw = E // NW
    iters = epw // C_CHUNK
    wid = lax.axis_index("s") * NC + lax.axis_index("c")
    base = wid * epw

    pltpu.sync_copy(b2a_hbm.at[pl.ds(base, epw)], ia_v)
    pltpu.sync_copy(b2revb_hbm.at[pl.ds(base, epw)], ir_v)

    def gathers(it, qa_v, qr_v, sem_a, sem_r):
        sl = pl.ds(it * C_CHUNK, C_CHUNK)
        return (pltpu.make_async_copy(agg_hbm.at[ia_v.at[sl]], qa_v, sem_a),
                pltpu.make_async_copy(msg_hbm.at[ir_v.at[sl]], qr_v, sem_r))

    def start(it, qa_v, qr_v, sem_a, sem_r):
        cpa, cpr = gathers(it, qa_v, qr_v, sem_a, sem_r)
        cpa.start()
        cpr.start()

    def finish(it, qa_v, qr_v, sem_a, sem_r):
        cpa, cpr = gathers(it, qa_v, qr_v, sem_a, sem_r)
        cpa.wait()
        cpr.wait()
        for r in range(C_CHUNK):
            for h in range(H // LANES):
                sl = pl.ds(h * LANES, LANES)
                o_v[r, sl] = qa_v[r, sl] - qr_v[r, sl]
        pltpu.sync_copy(o_v, out_hbm.at[pl.ds(base + it * C_CHUNK, C_CHUNK)])

    start(0, qa0_v, qr0_v, sem_a0, sem_r0)

    def body(p, carry):
        it0 = 2 * p
        start(it0 + 1, qa1_v, qr1_v, sem_a1, sem_r1)
        finish(it0, qa0_v, qr0_v, sem_a0, sem_r0)
        start(it0 + 2, qa0_v, qr0_v, sem_a0, sem_r0)
        finish(it0 + 1, qa1_v, qr1_v, sem_a1, sem_r1)
        return carry

    lax.fori_loop(0, iters // 2, body, 0)
    finish(iters - 1, qa0_v, qr0_v, sem_a0, sem_r0)


def _tc_input_proj(f_bonds, w_i):
    BE = 2000

    def body(x_ref, w_ref, inp_ref, msg_ref):
        acc = jnp.dot(x_ref[...], w_ref[...], preferred_element_type=jnp.float32)
        inp_ref[...] = acc
        msg_ref[...] = jnp.maximum(acc, 0.0)

    return pl.pallas_call(
        body,
        grid=(E // BE,),
        in_specs=[
            pl.BlockSpec((BE, BOND_FDIM), lambda i: (i, 0)),
            pl.BlockSpec((BOND_FDIM, H), lambda i: (0, 0)),
        ],
        out_specs=[
            pl.BlockSpec((BE, H), lambda i: (i, 0)),
            pl.BlockSpec((BE, H), lambda i: (i, 0)),
        ],
        out_shape=[
            jax.ShapeDtypeStruct((E, H), jnp.float32),
            jax.ShapeDtypeStruct((E, H), jnp.float32),
        ],
    )(f_bonds, w_i)


def _tc_update(pre, w_m, inp):
    BE = 2000

    def body(p_ref, w_ref, i_ref, o_ref):
        acc = jnp.dot(p_ref[...], w_ref[...], preferred_element_type=jnp.float32)
        o_ref[...] = jnp.maximum(i_ref[...] + acc, 0.0)

    return pl.pallas_call(
        body,
        grid=(E // BE,),
        in_specs=[
            pl.BlockSpec((BE, H), lambda i: (i, 0)),
            pl.BlockSpec((H, H), lambda i: (0, 0)),
            pl.BlockSpec((BE, H), lambda i: (i, 0)),
        ],
        out_specs=pl.BlockSpec((BE, H), lambda i: (i, 0)),
        out_shape=jax.ShapeDtypeStruct((E, H), jnp.float32),
    )(pre, w_m, inp)


def _tc_readout(f_atoms, a_msg, w_a1, w_a2, b_a, pool):
    def body(f_ref, m_ref, w1_ref, w2_ref, b_ref, p_ref, o_ref):
        h = jnp.dot(f_ref[...], w1_ref[...], preferred_element_type=jnp.float32)
        h = h + jnp.dot(m_ref[...], w2_ref[...], preferred_element_type=jnp.float32)
        h = jnp.maximum(h + b_ref[...], 0.0)
        o_ref[...] = jnp.dot(p_ref[...], h, preferred_element_type=jnp.float32)

    return pl.pallas_call(
        body,
        out_shape=jax.ShapeDtypeStruct((B, H), jnp.float32),
    )(f_atoms, a_msg, w_a1, w_a2, b_a, pool)


def kernel(f_atoms, f_bonds, a2b, b2a, b2revb, W_i, W_m, W_a, b_a):
    a2b_flat = jnp.pad(a2b, ((0, NPAD - N), (0, 0))).reshape(-1)
    w_a1 = W_a[:ATOM_FDIM]
    w_a2 = W_a[ATOM_FDIM:]
    b_a2 = b_a.reshape(1, H)
    apm = N // B
    row = jax.lax.broadcasted_iota(jnp.int32, (B, N), 0)
    col = jax.lax.broadcasted_iota(jnp.int32, (B, N), 1)
    pool = jnp.where(col // apm == row, 1.0 / apm, 0.0).astype(jnp.float32)

    inp, msg = _tc_input_proj(f_bonds, W_i)
    for _ in range(STEPS - 1):
        agg = _sc_gather_sum(msg, a2b_flat)
        pre = _sc_combine(agg, msg, b2a, b2revb)
        msg = _tc_update(pre, W_m, inp)
    a_msg = _sc_gather_sum(msg, a2b_flat)[:N]
    return _tc_readout(f_atoms, a_msg, w_a1, w_a2, b_a2, pool)


# speedup vs baseline: 1.4462x; 1.1781x over previous
import functools

import jax
import jax.numpy as jnp
from jax import lax
from jax.experimental import pallas as pl
from jax.experimental.pallas import tpu as pltpu
from jax.experimental.pallas import tpu_sc as plsc

N = 10000
E = 320000
MAX_NB = 32
ATOM_FDIM = 128
BOND_FDIM = 144
H = 128
STEPS = 3
B = 100

NC = 2
NS = 16
NW = NC * NS
LANES = 16

NPAD = 10368
A_CHUNK = 4
C_CHUNK = 80


def _sc_mesh():
    return plsc.VectorSubcoreMesh(
        core_axis_name="c", subcore_axis_name="s",
        num_cores=NC, num_subcores=NS,
    )


NBUF = 4


@functools.partial(
    pl.kernel,
    out_type=jax.ShapeDtypeStruct((NPAD, H), jnp.float32),
    mesh=_sc_mesh(),
    scratch_types=[
        pltpu.VMEM((NPAD // NW * MAX_NB,), jnp.int32),
        [pltpu.VMEM((A_CHUNK * MAX_NB, H), jnp.float32)] * NBUF,
        pltpu.VMEM((A_CHUNK, H), jnp.float32),
        [pltpu.SemaphoreType.DMA] * NBUF,
    ],
)
def _sc_gather_sum(msg_hbm, a2b_hbm, out_hbm, idx_v, rows_bufs, acc_v, sems):
    npw = NPAD // NW
    iters = npw // A_CHUNK
    nidx = A_CHUNK * MAX_NB
    wid = lax.axis_index("s") * NC + lax.axis_index("c")
    base_atom = wid * npw

    pltpu.sync_copy(a2b_hbm.at[pl.ds(base_atom * MAX_NB, npw * MAX_NB)], idx_v)

    def gather(it, b):
        return pltpu.make_async_copy(
            msg_hbm.at[idx_v.at[pl.ds(it * nidx, nidx)]],
            rows_bufs[b], sems[b])

    def reduce_store(it, b):
        rows_v = rows_bufs[b]
        for a in range(A_CHUNK):
            def knee(k, accs):
                return tuple(
                    accs[h] + rows_v[a * MAX_NB + k, pl.ds(h * LANES, LANES)]
                    for h in range(H // LANES))
            init = tuple(
                rows_v[a * MAX_NB, pl.ds(h * LANES, LANES)]
                for h in range(H // LANES))
            accs = lax.fori_loop(1, MAX_NB, knee, init)
            for h in range(H // LANES):
                acc_v[a, pl.ds(h * LANES, LANES)] = accs[h]
        pltpu.sync_copy(acc_v,
                        out_hbm.at[pl.ds(base_atom + it * A_CHUNK, A_CHUNK)])

    for b in range(NBUF - 1):
        gather(b, b).start()

    def body(g, carry):
        for j in range(NBUF):
            it = NBUF * g + j
            gather(it, j).wait()
            reduce_store(it, j)

            @pl.when(it + NBUF - 1 < iters)
            def _():
                gather(it + NBUF - 1, (j + NBUF - 1) % NBUF).start()
        return carry

    lax.fori_loop(0, iters // NBUF, body, 0)
    last = iters - 1
    gather(last, last % NBUF).wait()
    reduce_store(last, last % NBUF)


@functools.partial(
    pl.kernel,
    out_type=jax.ShapeDtypeStruct((E, H), jnp.float32),
    mesh=_sc_mesh(),
    scratch_types=[
        pltpu.VMEM((E // NW,), jnp.int32),
        pltpu.VMEM((E // NW,), jnp.int32),
        [pltpu.VMEM((C_CHUNK, H), jnp.float32)] * NBUF,
        [pltpu.VMEM((C_CHUNK, H), jnp.float32)] * NBUF,
        pltpu.VMEM((C_CHUNK, H), jnp.float32),
        [pltpu.SemaphoreType.DMA] * NBUF,
        [pltpu.SemaphoreType.DMA] * NBUF,
    ],
)
def _sc_combine(agg_hbm, msg_hbm, b2a_hbm, b2revb_hbm, out_hbm,
                ia_v, ir_v, qa_bufs, qr_bufs, o_v, sems_a, sems_r):
    epw = E // NW
    iters = epw // C_CHUNK
    wid = lax.axis_index("s") * NC + lax.axis_index("c")
    base = wid * epw

    pltpu.sync_copy(b2a_hbm.at[pl.ds(base, epw)], ia_v)
    pltpu.sync_copy(b2revb_hbm.at[pl.ds(base, epw)], ir_v)

    def gathers(it, b):
        sl = pl.ds(it * C_CHUNK, C_CHUNK)
        return (pltpu.make_async_copy(agg_hbm.at[ia_v.at[sl]],
                                      qa_bufs[b], sems_a[b]),
                pltpu.make_async_copy(msg_hbm.at[ir_v.at[sl]],
                                      qr_bufs[b], sems_r[b]))

    def start(it, b):
        cpa, cpr = gathers(it, b)
        cpa.start()
        cpr.start()

    def finish(it, b):
        cpa, cpr = gathers(it, b)
        cpa.wait()
        cpr.wait()
        qa_v, qr_v = qa_bufs[b], qr_bufs[b]

        def row(r, carry):
            for h in range(H // LANES):
                sl = pl.ds(h * LANES, LANES)
                o_v[r, sl] = qa_v[r, sl] - qr_v[r, sl]
            return carry

        lax.fori_loop(0, C_CHUNK, row, 0)
        pltpu.sync_copy(o_v, out_hbm.at[pl.ds(base + it * C_CHUNK, C_CHUNK)])

    for b in range(NBUF - 1):
        start(b, b)

    def body(g, carry):
        for j in range(NBUF):
            it = NBUF * g + j
            finish(it, j)

            @pl.when(it + NBUF - 1 < iters)
            def _():
                start(it + NBUF - 1, (j + NBUF - 1) % NBUF)
        return carry

    lax.fori_loop(0, iters // NBUF, body, 0)
    last = iters - 1
    finish(last, last % NBUF)


def _tc_input_proj(f_bonds, w_i):
    BE = 2000

    def body(x_ref, w_ref, inp_ref, msg_ref):
        acc = jnp.dot(x_ref[...], w_ref[...], preferred_element_type=jnp.float32)
        inp_ref[...] = acc
        msg_ref[...] = jnp.maximum(acc, 0.0)

    return pl.pallas_call(
        body,
        grid=(E // BE,),
        in_specs=[
            pl.BlockSpec((BE, BOND_FDIM), lambda i: (i, 0)),
            pl.BlockSpec((BOND_FDIM, H), lambda i: (0, 0)),
        ],
        out_specs=[
            pl.BlockSpec((BE, H), lambda i: (i, 0)),
            pl.BlockSpec((BE, H), lambda i: (i, 0)),
        ],
        out_shape=[
            jax.ShapeDtypeStruct((E, H), jnp.float32),
            jax.ShapeDtypeStruct((E, H), jnp.float32),
        ],
    )(f_bonds, w_i)


def _tc_update(pre, w_m, inp):
    BE = 2000

    def body(p_ref, w_ref, i_ref, o_ref):
        acc = jnp.dot(p_ref[...], w_ref[...], preferred_element_type=jnp.float32)
        o_ref[...] = jnp.maximum(i_ref[...] + acc, 0.0)

    return pl.pallas_call(
        body,
        grid=(E // BE,),
        in_specs=[
            pl.BlockSpec((BE, H), lambda i: (i, 0)),
            pl.BlockSpec((H, H), lambda i: (0, 0)),
            pl.BlockSpec((BE, H), lambda i: (i, 0)),
        ],
        out_specs=pl.BlockSpec((BE, H), lambda i: (i, 0)),
        out_shape=jax.ShapeDtypeStruct((E, H), jnp.float32),
    )(pre, w_m, inp)


def _tc_readout(f_atoms, a_msg, w_a1, w_a2, b_a, pool):
    def body(f_ref, m_ref, w1_ref, w2_ref, b_ref, p_ref, o_ref):
        h = jnp.dot(f_ref[...], w1_ref[...], preferred_element_type=jnp.float32)
        h = h + jnp.dot(m_ref[...], w2_ref[...], preferred_element_type=jnp.float32)
        h = jnp.maximum(h + b_ref[...], 0.0)
        o_ref[...] = jnp.dot(p_ref[...], h, preferred_element_type=jnp.float32)

    return pl.pallas_call(
        body,
        out_shape=jax.ShapeDtypeStruct((B, H), jnp.float32),
    )(f_atoms, a_msg, w_a1, w_a2, b_a, pool)


def kernel(f_atoms, f_bonds, a2b, b2a, b2revb, W_i, W_m, W_a, b_a):
    a2b_flat = jnp.pad(a2b, ((0, NPAD - N), (0, 0))).reshape(-1)
    w_a1 = W_a[:ATOM_FDIM]
    w_a2 = W_a[ATOM_FDIM:]
    b_a2 = b_a.reshape(1, H)
    apm = N // B
    row = jax.lax.broadcasted_iota(jnp.int32, (B, N), 0)
    col = jax.lax.broadcasted_iota(jnp.int32, (B, N), 1)
    pool = jnp.where(col // apm == row, 1.0 / apm, 0.0).astype(jnp.float32)

    inp, msg = _tc_input_proj(f_bonds, W_i)
    for _ in range(STEPS - 1):
        agg = _sc_gather_sum(msg, a2b_flat)
        pre = _sc_combine(agg, msg, b2a, b2revb)
        msg = _tc_update(pre, W_m, inp)
    a_msg = _sc_gather_sum(msg, a2b_flat)[:N]
    return _tc_readout(f_atoms, a_msg, w_a1, w_a2, b_a2, pool)


# speedup vs baseline: 1.4468x; 1.0004x over previous
import functools

import jax
import jax.numpy as jnp
from jax import lax
from jax.experimental import pallas as pl
from jax.experimental.pallas import tpu as pltpu
from jax.experimental.pallas import tpu_sc as plsc

N = 10000
E = 320000
MAX_NB = 32
ATOM_FDIM = 128
BOND_FDIM = 144
H = 128
STEPS = 3
B = 100

NC = 2
NS = 16
NW = NC * NS
LANES = 16

NPAD = 10368
A_CHUNK = 4
C_CHUNK = 80


def _sc_mesh():
    return plsc.VectorSubcoreMesh(
        core_axis_name="c", subcore_axis_name="s",
        num_cores=NC, num_subcores=NS,
    )


NBUF = 4


@functools.partial(
    pl.kernel,
    out_type=jax.ShapeDtypeStruct((NPAD, H), jnp.float32),
    mesh=_sc_mesh(),
    scratch_types=[
        pltpu.VMEM((NPAD // NW * MAX_NB,), jnp.int32),
        [pltpu.VMEM((A_CHUNK * MAX_NB, H), jnp.float32)] * NBUF,
        pltpu.VMEM((A_CHUNK, H), jnp.float32),
        [pltpu.SemaphoreType.DMA] * NBUF,
        [pltpu.SemaphoreType.DMA] * NBUF,
    ],
)
def _sc_gather_sum(msg_hbm, a2b_hbm, out_hbm, idx_v, rows_bufs, acc_v, sems,
                   sems2):
    npw = NPAD // NW
    iters = npw // A_CHUNK
    nidx = A_CHUNK * MAX_NB
    wid = lax.axis_index("s") * NC + lax.axis_index("c")
    base_atom = wid * npw

    pltpu.sync_copy(a2b_hbm.at[pl.ds(base_atom * MAX_NB, npw * MAX_NB)], idx_v)

    half = nidx // 2

    def gather_pair(it, b):
        return (
            pltpu.make_async_copy(
                msg_hbm.at[idx_v.at[pl.ds(it * nidx, half)]],
                rows_bufs[b].at[pl.ds(0, half)], sems[b]),
            pltpu.make_async_copy(
                msg_hbm.at[idx_v.at[pl.ds(it * nidx + half, half)]],
                rows_bufs[b].at[pl.ds(half, half)], sems2[b]),
        )

    def gather_start(it, b):
        cp0, cp1 = gather_pair(it, b)
        cp0.start()
        cp1.start()

    def gather_wait(it, b):
        cp0, cp1 = gather_pair(it, b)
        cp0.wait()
        cp1.wait()

    def reduce_store(it, b):
        rows_v = rows_bufs[b]
        for a in range(A_CHUNK):
            def knee(k, accs):
                return tuple(
                    accs[h] + rows_v[a * MAX_NB + k, pl.ds(h * LANES, LANES)]
                    for h in range(H // LANES))
            init = tuple(
                rows_v[a * MAX_NB, pl.ds(h * LANES, LANES)]
                for h in range(H // LANES))
            accs = lax.fori_loop(1, MAX_NB, knee, init)
            for h in range(H // LANES):
                acc_v[a, pl.ds(h * LANES, LANES)] = accs[h]
        pltpu.sync_copy(acc_v,
                        out_hbm.at[pl.ds(base_atom + it * A_CHUNK, A_CHUNK)])

    for b in range(NBUF - 1):
        gather_start(b, b)

    def body(g, carry):
        for j in range(NBUF):
            it = NBUF * g + j
            gather_wait(it, j)
            reduce_store(it, j)

            @pl.when(it + NBUF - 1 < iters)
            def _():
                gather_start(it + NBUF - 1, (j + NBUF - 1) % NBUF)
        return carry

    lax.fori_loop(0, iters // NBUF, body, 0)
    last = iters - 1
    gather_wait(last, last % NBUF)
    reduce_store(last, last % NBUF)


@functools.partial(
    pl.kernel,
    out_type=jax.ShapeDtypeStruct((E, H), jnp.float32),
    mesh=_sc_mesh(),
    scratch_types=[
        pltpu.VMEM((E // NW,), jnp.int32),
        pltpu.VMEM((E // NW,), jnp.int32),
        [pltpu.VMEM((C_CHUNK, H), jnp.float32)] * NBUF,
        [pltpu.VMEM((C_CHUNK, H), jnp.float32)] * NBUF,
        pltpu.VMEM((C_CHUNK, H), jnp.float32),
        [pltpu.SemaphoreType.DMA] * NBUF,
        [pltpu.SemaphoreType.DMA] * NBUF,
    ],
)
def _sc_combine(agg_hbm, msg_hbm, b2a_hbm, b2revb_hbm, out_hbm,
                ia_v, ir_v, qa_bufs, qr_bufs, o_v, sems_a, sems_r):
    epw = E // NW
    iters = epw // C_CHUNK
    wid = lax.axis_index("s") * NC + lax.axis_index("c")
    base = wid * epw

    pltpu.sync_copy(b2a_hbm.at[pl.ds(base, epw)], ia_v)
    pltpu.sync_copy(b2revb_hbm.at[pl.ds(base, epw)], ir_v)

    def gathers(it, b):
        sl = pl.ds(it * C_CHUNK, C_CHUNK)
        return (pltpu.make_async_copy(agg_hbm.at[ia_v.at[sl]],
                                      qa_bufs[b], sems_a[b]),
                pltpu.make_async_copy(msg_hbm.at[ir_v.at[sl]],
                                      qr_bufs[b], sems_r[b]))

    def start(it, b):
        cpa, cpr = gathers(it, b)
        cpa.start()
        cpr.start()

    def finish(it, b):
        cpa, cpr = gathers(it, b)
        cpa.wait()
        cpr.wait()
        qa_v, qr_v = qa_bufs[b], qr_bufs[b]

        def row(r, carry):
            for h in range(H // LANES):
                sl = pl.ds(h * LANES, LANES)
                o_v[r, sl] = qa_v[r, sl] - qr_v[r, sl]
            return carry

        lax.fori_loop(0, C_CHUNK, row, 0)
        pltpu.sync_copy(o_v, out_hbm.at[pl.ds(base + it * C_CHUNK, C_CHUNK)])

    for b in range(NBUF - 1):
        start(b, b)

    def body(g, carry):
        for j in range(NBUF):
            it = NBUF * g + j
            finish(it, j)

            @pl.when(it + NBUF - 1 < iters)
            def _():
                start(it + NBUF - 1, (j + NBUF - 1) % NBUF)
        return carry

    lax.fori_loop(0, iters // NBUF, body, 0)
    last = iters - 1
    finish(last, last % NBUF)


def _tc_input_proj(f_bonds, w_i):
    BE = 2000

    def body(x_ref, w_ref, inp_ref, msg_ref):
        acc = jnp.dot(x_ref[...], w_ref[...], preferred_element_type=jnp.float32)
        inp_ref[...] = acc
        msg_ref[...] = jnp.maximum(acc, 0.0)

    return pl.pallas_call(
        body,
        grid=(E // BE,),
        in_specs=[
            pl.BlockSpec((BE, BOND_FDIM), lambda i: (i, 0)),
            pl.BlockSpec((BOND_FDIM, H), lambda i: (0, 0)),
        ],
        out_specs=[
            pl.BlockSpec((BE, H), lambda i: (i, 0)),
            pl.BlockSpec((BE, H), lambda i: (i, 0)),
        ],
        out_shape=[
            jax.ShapeDtypeStruct((E, H), jnp.float32),
            jax.ShapeDtypeStruct((E, H), jnp.float32),
        ],
    )(f_bonds, w_i)


def _tc_update(pre, w_m, inp):
    BE = 2000

    def body(p_ref, w_ref, i_ref, o_ref):
        acc = jnp.dot(p_ref[...], w_ref[...], preferred_element_type=jnp.float32)
        o_ref[...] = jnp.maximum(i_ref[...] + acc, 0.0)

    return pl.pallas_call(
        body,
        grid=(E // BE,),
        in_specs=[
            pl.BlockSpec((BE, H), lambda i: (i, 0)),
            pl.BlockSpec((H, H), lambda i: (0, 0)),
            pl.BlockSpec((BE, H), lambda i: (i, 0)),
        ],
        out_specs=pl.BlockSpec((BE, H), lambda i: (i, 0)),
        out_shape=jax.ShapeDtypeStruct((E, H), jnp.float32),
    )(pre, w_m, inp)


def _tc_readout(f_atoms, a_msg, w_a1, w_a2, b_a, pool):
    def body(f_ref, m_ref, w1_ref, w2_ref, b_ref, p_ref, o_ref):
        h = jnp.dot(f_ref[...], w1_ref[...], preferred_element_type=jnp.float32)
        h = h + jnp.dot(m_ref[...], w2_ref[...], preferred_element_type=jnp.float32)
        h = jnp.maximum(h + b_ref[...], 0.0)
        o_ref[...] = jnp.dot(p_ref[...], h, preferred_element_type=jnp.float32)

    return pl.pallas_call(
        body,
        out_shape=jax.ShapeDtypeStruct((B, H), jnp.float32),
    )(f_atoms, a_msg, w_a1, w_a2, b_a, pool)


def kernel(f_atoms, f_bonds, a2b, b2a, b2revb, W_i, W_m, W_a, b_a):
    a2b_flat = jnp.pad(a2b, ((0, NPAD - N), (0, 0))).reshape(-1)
    w_a1 = W_a[:ATOM_FDIM]
    w_a2 = W_a[ATOM_FDIM:]
    b_a2 = b_a.reshape(1, H)
    apm = N // B
    row = jax.lax.broadcasted_iota(jnp.int32, (B, N), 0)
    col = jax.lax.broadcasted_iota(jnp.int32, (B, N), 1)
    pool = jnp.where(col // apm == row, 1.0 / apm, 0.0).astype(jnp.float32)

    inp, msg = _tc_input_proj(f_bonds, W_i)
    for _ in range(STEPS - 1):
        agg = _sc_gather_sum(msg, a2b_flat)
        pre = _sc_combine(agg, msg, b2a, b2revb)
        msg = _tc_update(pre, W_m, inp)
    a_msg = _sc_gather_sum(msg, a2b_flat)[:N]
    return _tc_readout(f_atoms, a_msg, w_a1, w_a2, b_a2, pool)
